# baseline (speedup 1.0000x reference)
import jax
import jax.numpy as jnp
from jax.experimental import pallas as pl
from jax.experimental.pallas import tpu as pltpu

B = 16
G = 15000
DVEC = 200
DM = 32
NBITS = 30
RANKS = (48000, 96000, 144000, 192000)
KEY_HI = 0x3F800000


def _prep_kernel(x_ref, be_ref, wgp_ref, bgp_ref, wcp_ref, bcp_ref,
                 thr_ref, wfold_ref, w2_ref, c512_ref):
    keys = jax.lax.bitcast_convert_type(x_ref[...], jnp.int32)

    def count_le(t):
        return jnp.sum((keys <= t).astype(jnp.int32))

    def body(_, carry):
        new = []
        for j in range(4):
            lo, hi = carry[2 * j], carry[2 * j + 1]
            mid = lo + (hi - lo) // 2
            ge = count_le(mid) >= (RANKS[j] + 1)
            new.append(jnp.where(ge, lo, mid + 1))
            new.append(jnp.where(ge, mid, hi))
        return tuple(new)

    init = ()
    for j in range(4):
        init += (jnp.int32(0), jnp.int32(KEY_HI))
    carry = jax.lax.fori_loop(0, NBITS, body, init)
    tadj = []
    for j in range(4):
        v = carry[2 * j]
        strict = count_le(v - 1) >= RANKS[j]
        tadj.append(jnp.where(strict, v - 1, v))

    l64 = jax.lax.broadcasted_iota(jnp.int32, (1, 64), 1)
    jd = l64 // 16
    thr_ref[...] = jnp.where(
        jd == 0, tadj[0],
        jnp.where(jd == 1, tadj[1], jnp.where(jd == 2, tadj[2], tadj[3])))

    wcp_top = wcp_ref[:DM, :]
    wcp_bot = wcp_ref[DM:, :]
    wfold_ref[...] = jnp.dot(wgp_ref[...], wcp_top,
                             preferred_element_type=jnp.float32,
                  precision=jax.lax.Precision.HIGHEST)
    bt = jnp.dot(be_ref[...], wcp_bot, preferred_element_type=jnp.float32,
                  precision=jax.lax.Precision.HIGHEST)
    dbt = bt[1:, :] - bt[:-1, :]

    dbt_exp = jnp.broadcast_to(dbt[:, None, :], (4, 16, DM)).reshape(64, DM)
    dbt_exp = jnp.broadcast_to(dbt_exp[:, None, :],
                               (64, 16, DM)).reshape(64, 16 * DM)
    r = jax.lax.broadcasted_iota(jnp.int32, (64, 16 * DM), 0)
    c = jax.lax.broadcasted_iota(jnp.int32, (64, 16 * DM), 1)
    w_top = jnp.where((r % 16) == (c // DM), dbt_exp, 0.0)
    rb = jax.lax.broadcasted_iota(jnp.int32, (DM, 16 * DM), 0)
    cb = jax.lax.broadcasted_iota(jnp.int32, (DM, 16 * DM), 1)
    w_bot = jnp.where(rb == (cb % DM), 1.0, 0.0)
    w2_ref[...] = jnp.concatenate([w_top, w_bot], axis=0)

    cvec = (jnp.dot(bgp_ref[...][None, :], wcp_top,
                    preferred_element_type=jnp.float32,
                  precision=jax.lax.Precision.HIGHEST)
            + bcp_ref[...][None, :] + bt[0:1, :])
    c512_ref[...] = jnp.broadcast_to(cvec[:, None, :],
                                     (1, 16, DM)).reshape(1, 16 * DM)


def _fused_kernel(thr_ref, w2_ref, c512_ref, keys4_ref, gv_ref, wfold_ref,
                  out_ref):
    ind = (keys4_ref[...] > thr_ref[...]).astype(jnp.float32)
    base = jnp.dot(gv_ref[...], wfold_ref[...],
                   preferred_element_type=jnp.float32,
                  precision=jax.lax.Precision.HIGHEST)
    a2 = jnp.concatenate([ind, base], axis=1)
    res = jnp.dot(a2, w2_ref[...],
                  preferred_element_type=jnp.float32,
                  precision=jax.lax.Precision.HIGHEST) + c512_ref[...]
    for b in range(B):
        out_ref[b] = res[:, b * DM:(b + 1) * DM]


def kernel(gene_expression, genevec_embeddings, W_gp, b_gp, bin_emb, W_cp,
           b_cp):
    thr64, wfold, w2, c512 = pl.pallas_call(
        _prep_kernel,
        out_shape=[
            jax.ShapeDtypeStruct((1, 64), jnp.int32),
            jax.ShapeDtypeStruct((DVEC, DM), jnp.float32),
            jax.ShapeDtypeStruct((96, 16 * DM), jnp.float32),
            jax.ShapeDtypeStruct((1, 16 * DM), jnp.float32),
        ],
        in_specs=[pl.BlockSpec(memory_space=pltpu.VMEM)] * 6,
        out_specs=[pl.BlockSpec(memory_space=pltpu.VMEM)] * 4,
    )(gene_expression, bin_emb, W_gp, b_gp, W_cp, b_cp)

    keysT = jax.lax.bitcast_convert_type(gene_expression.T, jnp.int32)
    keys4 = jnp.concatenate([keysT] * 4, axis=1)

    TG = 1024
    grid = (pl.cdiv(G, TG),)
    out = pl.pallas_call(
        _fused_kernel,
        grid=grid,
        out_shape=jax.ShapeDtypeStruct((B, G, DM), jnp.float32),
        in_specs=[
            pl.BlockSpec((1, 64), lambda i: (0, 0)),
            pl.BlockSpec((96, 16 * DM), lambda i: (0, 0)),
            pl.BlockSpec((1, 16 * DM), lambda i: (0, 0)),
            pl.BlockSpec((TG, 64), lambda i: (i, 0)),
            pl.BlockSpec((TG, DVEC), lambda i: (i, 0)),
            pl.BlockSpec((DVEC, DM), lambda i: (0, 0)),
        ],
        out_specs=pl.BlockSpec((B, TG, DM), lambda i: (0, i, 0)),
        compiler_params=pltpu.CompilerParams(
            dimension_semantics=("arbitrary",)),
    )(thr64, w2, c512, keys4, genevec_embeddings, wfold)
    return out


# speedup vs baseline: 1.1472x; 1.1472x over previous
import jax
import jax.numpy as jnp
from jax.experimental import pallas as pl
from jax.experimental.pallas import tpu as pltpu

B = 16
G = 15000
DVEC = 200
DM = 32
NBITS = 30
RANKS = (48000, 96000, 144000, 192000)
KEY_HI = 0x3F800000


def _prep_kernel(x_ref, be_ref, bgp_ref, wcp_ref, bcp_ref,
                 thr_ref, w2_ref, c512_ref):
    keys = jax.lax.bitcast_convert_type(x_ref[...], jnp.int32)

    def count_le(t):
        return jnp.sum((keys <= t).astype(jnp.int32))

    def body(_, carry):
        new = []
        for j in range(4):
            lo, hi = carry[2 * j], carry[2 * j + 1]
            mid = lo + (hi - lo) // 2
            ge = count_le(mid) >= (RANKS[j] + 1)
            new.append(jnp.where(ge, lo, mid + 1))
            new.append(jnp.where(ge, mid, hi))
        return tuple(new)

    init = ()
    for j in range(4):
        init += (jnp.int32(0), jnp.int32(KEY_HI))
    carry = jax.lax.fori_loop(0, NBITS, body, init)
    tadj = []
    for j in range(4):
        v = carry[2 * j]
        strict = count_le(v - 1) >= RANKS[j]
        tadj.append(jnp.where(strict, v - 1, v))

    l64 = jax.lax.broadcasted_iota(jnp.int32, (1, 64), 1)
    jd = l64 // 16
    thr_ref[...] = jnp.where(
        jd == 0, tadj[0],
        jnp.where(jd == 1, tadj[1], jnp.where(jd == 2, tadj[2], tadj[3])))

    wcp_top = wcp_ref[:DM, :]
    wcp_bot = wcp_ref[DM:, :]
    bt = jnp.dot(be_ref[...], wcp_bot, preferred_element_type=jnp.float32)
    dbt = bt[1:, :] - bt[:-1, :]

    dbt_exp = jnp.broadcast_to(dbt[:, None, :], (4, 16, DM)).reshape(64, DM)
    dbt_exp = jnp.broadcast_to(dbt_exp[:, None, :],
                               (64, 16, DM)).reshape(64, 16 * DM)
    r = jax.lax.broadcasted_iota(jnp.int32, (64, 16 * DM), 0)
    c = jax.lax.broadcasted_iota(jnp.int32, (64, 16 * DM), 1)
    w_top = jnp.where((r % 16) == (c // DM), dbt_exp, 0.0)
    w_bot = jnp.broadcast_to(wcp_top[:, None, :],
                             (DM, 16, DM)).reshape(DM, 16 * DM)
    w2_ref[...] = jnp.concatenate([w_top, w_bot], axis=0)

    cvec = (jnp.dot(bgp_ref[...][None, :], wcp_top,
                    preferred_element_type=jnp.float32)
            + bcp_ref[...][None, :] + bt[0:1, :])
    c512_ref[...] = jnp.broadcast_to(cvec[:, None, :],
                                     (1, 16, DM)).reshape(1, 16 * DM)


def _fused_kernel(thr_ref, w2_ref, c512_ref, keys4_ref, gv_ref, wgp_ref,
                  out_ref):
    ind = (keys4_ref[...] > thr_ref[...]).astype(jnp.float32)
    proj = jnp.dot(gv_ref[...], wgp_ref[...],
                   preferred_element_type=jnp.float32)
    a2 = jnp.concatenate([ind, proj], axis=1)
    res = jnp.dot(a2, w2_ref[...],
                  preferred_element_type=jnp.float32) + c512_ref[...]
    for b in range(B):
        out_ref[b] = res[:, b * DM:(b + 1) * DM]


def kernel(gene_expression, genevec_embeddings, W_gp, b_gp, bin_emb, W_cp,
           b_cp):
    thr64, w2, c512 = pl.pallas_call(
        _prep_kernel,
        out_shape=[
            jax.ShapeDtypeStruct((1, 64), jnp.int32),
            jax.ShapeDtypeStruct((96, 16 * DM), jnp.float32),
            jax.ShapeDtypeStruct((1, 16 * DM), jnp.float32),
        ],
        in_specs=[pl.BlockSpec(memory_space=pltpu.VMEM)] * 5,
        out_specs=[pl.BlockSpec(memory_space=pltpu.VMEM)] * 3,
    )(gene_expression, bin_emb, b_gp, W_cp, b_cp)

    keysT = jax.lax.bitcast_convert_type(gene_expression.T, jnp.int32)
    keys4 = jnp.concatenate([keysT] * 4, axis=1)

    TG = 1024
    grid = (pl.cdiv(G, TG),)
    out = pl.pallas_call(
        _fused_kernel,
        grid=grid,
        out_shape=jax.ShapeDtypeStruct((B, G, DM), jnp.float32),
        in_specs=[
            pl.BlockSpec((1, 64), lambda i: (0, 0)),
            pl.BlockSpec((96, 16 * DM), lambda i: (0, 0)),
            pl.BlockSpec((1, 16 * DM), lambda i: (0, 0)),
            pl.BlockSpec((TG, 64), lambda i: (i, 0)),
            pl.BlockSpec((TG, DVEC), lambda i: (i, 0)),
            pl.BlockSpec((DVEC, DM), lambda i: (0, 0)),
        ],
        out_specs=pl.BlockSpec((B, TG, DM), lambda i: (0, i, 0)),
        compiler_params=pltpu.CompilerParams(
            dimension_semantics=("arbitrary",)),
    )(thr64, w2, c512, keys4, genevec_embeddings, W_gp)
    return out


# speedup vs baseline: 1.2455x; 1.0856x over previous
import jax
import jax.numpy as jnp
from jax.experimental import pallas as pl
from jax.experimental.pallas import tpu as pltpu

B = 16
G = 15000
DVEC = 200
DM = 32
NBITS = 30
RANKS = (48000, 96000, 144000, 192000)
KEY_HI = 0x3F800000


def _prep_kernel(x_ref, be_ref, bgp_ref, wcp_ref, bcp_ref,
                 thr_ref, w2_ref, c512_ref):
    keys = jax.lax.bitcast_convert_type(x_ref[...], jnp.int32)

    def count_le(t):
        return jnp.sum((keys <= t).astype(jnp.int32))

    def body(_, carry):
        new = []
        for j in range(4):
            lo, hi = carry[2 * j], carry[2 * j + 1]
            mid = lo + (hi - lo) // 2
            ge = count_le(mid) >= (RANKS[j] + 1)
            new.append(jnp.where(ge, lo, mid + 1))
            new.append(jnp.where(ge, mid, hi))
        return tuple(new)

    init = ()
    for j in range(4):
        init += (jnp.int32(0), jnp.int32(KEY_HI))
    carry = jax.lax.fori_loop(0, NBITS, body, init)
    tadj = []
    for j in range(4):
        v = carry[2 * j]
        strict = count_le(v - 1) >= RANKS[j]
        tadj.append(jnp.where(strict, v - 1, v))

    l64 = jax.lax.broadcasted_iota(jnp.int32, (1, 64), 1)
    jd = l64 // 16
    thr_ref[...] = jnp.where(
        jd == 0, tadj[0],
        jnp.where(jd == 1, tadj[1], jnp.where(jd == 2, tadj[2], tadj[3])))

    wcp_top = wcp_ref[:DM, :]
    wcp_bot = wcp_ref[DM:, :]
    bt = jnp.dot(be_ref[...], wcp_bot, preferred_element_type=jnp.float32)
    dbt = bt[1:, :] - bt[:-1, :]

    dbt_exp = jnp.broadcast_to(dbt[:, None, :], (4, 16, DM)).reshape(64, DM)
    dbt_exp = jnp.broadcast_to(dbt_exp[:, None, :],
                               (64, 16, DM)).reshape(64, 16 * DM)
    r = jax.lax.broadcasted_iota(jnp.int32, (64, 16 * DM), 0)
    c = jax.lax.broadcasted_iota(jnp.int32, (64, 16 * DM), 1)
    w_top = jnp.where((r % 16) == (c // DM), dbt_exp, 0.0)
    w_bot = jnp.broadcast_to(wcp_top[:, None, :],
                             (DM, 16, DM)).reshape(DM, 16 * DM)
    w2_ref[...] = jnp.concatenate([w_top, w_bot], axis=0)

    cvec = (jnp.dot(bgp_ref[...][None, :], wcp_top,
                    preferred_element_type=jnp.float32)
            + bcp_ref[...][None, :] + bt[0:1, :])
    c512_ref[...] = jnp.broadcast_to(cvec[:, None, :],
                                     (1, 16, DM)).reshape(1, 16 * DM)


def _fused_kernel(thr_ref, w2_ref, c512_ref, keysT_ref, gv_ref, wgp_ref,
                  out_ref):
    kT = keysT_ref[...]
    keys4 = jnp.concatenate([kT, kT, kT, kT], axis=1)
    ind = (keys4 > thr_ref[...]).astype(jnp.float32)
    proj = jnp.dot(gv_ref[...], wgp_ref[...],
                   preferred_element_type=jnp.float32)
    a2 = jnp.concatenate([ind, proj], axis=1)
    res = jnp.dot(a2, w2_ref[...],
                  preferred_element_type=jnp.float32) + c512_ref[...]
    for b in range(B):
        out_ref[b] = res[:, b * DM:(b + 1) * DM]


def kernel(gene_expression, genevec_embeddings, W_gp, b_gp, bin_emb, W_cp,
           b_cp):
    thr64, w2, c512 = pl.pallas_call(
        _prep_kernel,
        out_shape=[
            jax.ShapeDtypeStruct((1, 64), jnp.int32),
            jax.ShapeDtypeStruct((96, 16 * DM), jnp.float32),
            jax.ShapeDtypeStruct((1, 16 * DM), jnp.float32),
        ],
        in_specs=[pl.BlockSpec(memory_space=pltpu.VMEM)] * 5,
        out_specs=[pl.BlockSpec(memory_space=pltpu.VMEM)] * 3,
    )(gene_expression, bin_emb, b_gp, W_cp, b_cp)

    keysT = jax.lax.bitcast_convert_type(gene_expression.T, jnp.int32)

    TG = 1024
    grid = (pl.cdiv(G, TG),)
    out = pl.pallas_call(
        _fused_kernel,
        grid=grid,
        out_shape=jax.ShapeDtypeStruct((B, G, DM), jnp.float32),
        in_specs=[
            pl.BlockSpec((1, 64), lambda i: (0, 0)),
            pl.BlockSpec((96, 16 * DM), lambda i: (0, 0)),
            pl.BlockSpec((1, 16 * DM), lambda i: (0, 0)),
            pl.BlockSpec((TG, 16), lambda i: (i, 0)),
            pl.BlockSpec((TG, DVEC), lambda i: (i, 0)),
            pl.BlockSpec((DVEC, DM), lambda i: (0, 0)),
        ],
        out_specs=pl.BlockSpec((B, TG, DM), lambda i: (0, i, 0)),
        compiler_params=pltpu.CompilerParams(
            dimension_semantics=("arbitrary",)),
    )(thr64, w2, c512, keysT, genevec_embeddings, W_gp)
    return out


# speedup vs baseline: 1.2525x; 1.0057x over previous
import jax
import jax.numpy as jnp
from jax.experimental import pallas as pl
from jax.experimental.pallas import tpu as pltpu

B = 16
G = 15000
DVEC = 200
DM = 32
NBITS = 30
RANKS = (48000, 96000, 144000, 192000)
KEY_HI = 0x3F800000


def _prep_kernel(x_ref, be_ref, bgp_ref, wcp_ref, bcp_ref,
                 thr_ref, w2_ref, c512_ref):
    keys = jax.lax.bitcast_convert_type(x_ref[...], jnp.int32)

    def count_le(t):
        return jnp.sum((keys <= t).astype(jnp.int32))

    def body(_, carry):
        new = []
        for j in range(4):
            lo, hi = carry[2 * j], carry[2 * j + 1]
            mid = lo + (hi - lo) // 2
            ge = count_le(mid) >= (RANKS[j] + 1)
            new.append(jnp.where(ge, lo, mid + 1))
            new.append(jnp.where(ge, mid, hi))
        return tuple(new)

    init = ()
    for j in range(4):
        init += (jnp.int32(0), jnp.int32(KEY_HI))
    carry = jax.lax.fori_loop(0, NBITS, body, init)
    tadj = []
    for j in range(4):
        v = carry[2 * j]
        strict = count_le(v - 1) >= RANKS[j]
        tadj.append(jnp.where(strict, v - 1, v))

    l64 = jax.lax.broadcasted_iota(jnp.int32, (1, 64), 1)
    jd = l64 // 16
    thr_ref[...] = jnp.where(
        jd == 0, tadj[0],
        jnp.where(jd == 1, tadj[1], jnp.where(jd == 2, tadj[2], tadj[3])))

    wcp_top = wcp_ref[:DM, :]
    wcp_bot = wcp_ref[DM:, :]
    bt = jnp.dot(be_ref[...], wcp_bot, preferred_element_type=jnp.float32)
    dbt = bt[1:, :] - bt[:-1, :]

    dbt_exp = jnp.broadcast_to(dbt[:, None, :], (4, 16, DM)).reshape(64, DM)
    dbt_exp = jnp.broadcast_to(dbt_exp[:, None, :],
                               (64, 16, DM)).reshape(64, 16 * DM)
    r = jax.lax.broadcasted_iota(jnp.int32, (64, 16 * DM), 0)
    c = jax.lax.broadcasted_iota(jnp.int32, (64, 16 * DM), 1)
    w_top = jnp.where((r % 16) == (c // DM), dbt_exp, 0.0)
    w_bot = jnp.broadcast_to(wcp_top[:, None, :],
                             (DM, 16, DM)).reshape(DM, 16 * DM)
    w2_ref[...] = jnp.concatenate([w_top, w_bot], axis=0)

    cvec = (jnp.dot(bgp_ref[...][None, :], wcp_top,
                    preferred_element_type=jnp.float32)
            + bcp_ref[...][None, :] + bt[0:1, :])
    c512_ref[...] = jnp.broadcast_to(cvec[:, None, :],
                                     (1, 16, DM)).reshape(1, 16 * DM)


def _fused_kernel(thr_ref, w2_ref, c512_ref, keysT_ref, gv_ref, wgp_ref,
                  out_ref):
    kT = keysT_ref[...]
    keys4 = jnp.concatenate([kT, kT, kT, kT], axis=1)
    ind = (keys4 > thr_ref[...]).astype(jnp.float32)
    proj = jnp.dot(gv_ref[...], wgp_ref[...],
                   preferred_element_type=jnp.float32)
    a2 = jnp.concatenate([ind, proj], axis=1)
    res = jnp.dot(a2, w2_ref[...],
                  preferred_element_type=jnp.float32) + c512_ref[...]
    for b in range(B):
        out_ref[b] = res[:, b * DM:(b + 1) * DM]


def kernel(gene_expression, genevec_embeddings, W_gp, b_gp, bin_emb, W_cp,
           b_cp):
    thr64, w2, c512 = pl.pallas_call(
        _prep_kernel,
        out_shape=[
            jax.ShapeDtypeStruct((1, 64), jnp.int32),
            jax.ShapeDtypeStruct((96, 16 * DM), jnp.float32),
            jax.ShapeDtypeStruct((1, 16 * DM), jnp.float32),
        ],
        in_specs=[pl.BlockSpec(memory_space=pltpu.VMEM)] * 5,
        out_specs=[pl.BlockSpec(memory_space=pltpu.VMEM)] * 3,
    )(gene_expression, bin_emb, b_gp, W_cp, b_cp)

    keysT = jax.lax.bitcast_convert_type(gene_expression.T, jnp.int32)

    TG = 2048
    grid = (pl.cdiv(G, TG),)
    out = pl.pallas_call(
        _fused_kernel,
        grid=grid,
        out_shape=jax.ShapeDtypeStruct((B, G, DM), jnp.float32),
        in_specs=[
            pl.BlockSpec((1, 64), lambda i: (0, 0)),
            pl.BlockSpec((96, 16 * DM), lambda i: (0, 0)),
            pl.BlockSpec((1, 16 * DM), lambda i: (0, 0)),
            pl.BlockSpec((TG, 16), lambda i: (i, 0)),
            pl.BlockSpec((TG, DVEC), lambda i: (i, 0)),
            pl.BlockSpec((DVEC, DM), lambda i: (0, 0)),
        ],
        out_specs=pl.BlockSpec((B, TG, DM), lambda i: (0, i, 0)),
        compiler_params=pltpu.CompilerParams(
            dimension_semantics=("arbitrary",)),
    )(thr64, w2, c512, keysT, genevec_embeddings, W_gp)
    return out
